# baseline (speedup 1.0000x reference)
import functools

import jax
import jax.numpy as jnp
from jax import lax
from jax.experimental import pallas as pl
from jax.experimental.pallas import tpu as pltpu
from jax.experimental.pallas import tpu_sc as plsc

_C = 14
_D = 64
_PAD = 15
_F = 128
_ZF = 96
_R = 192
_NW = 32
_B = 4096


def _pack_body(a_ref, ap_ref, s_ref, out_ref):
    f32 = jnp.float32
    i32 = jnp.int32
    a = a_ref[...]
    ap = ap_ref[...]
    s = s_ref[...]
    af = a * 16 + ap
    npad = jnp.dot((ap == _PAD).astype(f32), jnp.ones((_C, 1), f32),
                   preferred_element_type=f32)
    recip = 1.0 / ((_C - npad) + 1e-10)
    rbits = lax.bitcast_convert_type(recip, i32)
    out_ref[...] = jnp.concatenate([af, rbits, s], axis=1)


def _sc_hist(n, x_hbm, feat_hbm, x_v, feat_v):
    f32 = jnp.float32
    i32 = jnp.int32
    nchunks = n // _R
    kmax = (nchunks + _NW - 1) // _NW
    wid = lax.axis_index("s") * 2 + lax.axis_index("c")
    iota = lax.iota(i32, 16)
    ones = jnp.ones((16,), f32)
    zeros = jnp.zeros((16,), f32)


    def chunk_body(k, carry):
        cid = wid + k * _NW

        @pl.when(cid < nchunks)
        def _():
            base = pl.multiple_of(cid * _R, 64)
            pltpu.sync_copy(x_hbm.at[pl.ds(base, _R)], x_v)

            def zbody(r, c2):
                for u in range(_ZF // 16):
                    feat_v[r, pl.ds(u * 16, 16)] = zeros
                return c2

            lax.fori_loop(0, _R, zbody, 0)

            def gbody(g, c2):
                rows0 = g * 16 + iota
                valid = rows0 < _R
                rows = jnp.minimum(rows0, _R - 1)
                col0 = iota * 0
                recip = plsc.bitcast(
                    plsc.load_gather(x_v, [rows, col0 + _C]), f32)
                for c in range(_C):
                    afc = plsc.load_gather(x_v, [rows, col0 + c])
                    ac = lax.bitwise_and(lax.shift_right_logical(afc, 4), 63)
                    apc = lax.bitwise_and(afc, 15)
                    plsc.addupdate_scatter(feat_v, [rows, ac], recip,
                                           mask=valid & (apc != _PAD))
                    plsc.addupdate_scatter(feat_v, [rows, apc + 40],
                                           recip, mask=valid)
                sv = lax.bitwise_and(
                    plsc.load_gather(x_v, [rows, col0 + _C + 1]), 31)
                plsc.addupdate_scatter(feat_v, [rows, sv + 56], ones,
                                       mask=valid)
                return c2

            lax.fori_loop(0, (_R + 15) // 16, gbody, 0)
            pltpu.sync_copy(feat_v, feat_hbm.at[pl.ds(base, _R)])

        return carry

    lax.fori_loop(0, kmax, chunk_body, 0)


def _tc_body(feat_ref, rp_ref, t_ref, st_ref, out_ref):
    f32 = jnp.float32
    full = jnp.dot(feat_ref[...], t_ref[...], preferred_element_type=f32)
    rp = rp_ref[...].astype(f32)
    rpb = jnp.dot(rp, jnp.ones((1, 256), f32), preferred_element_type=f32)
    oh = (rpb == lax.broadcasted_iota(jnp.int32, (1, 256), 1).astype(f32))
    sinrows = jnp.dot(oh.astype(f32), st_ref[...], preferred_element_type=f32)
    out_ref[:, 0:_D] = full[:, 0:_D] + sinrows
    out_ref[:, _D:2 * _D] = full[:, _D:2 * _D]


def kernel(S, RP, A, AP, res_table, atom_table, atom_pos_table):
    f32 = jnp.float32
    i32 = jnp.int32
    n = S.shape[0]
    assert _R % 8 == 0 and (_R // 8) % 8 == 0 and _B % 64 == 0
    npd = (n + _R - 1) // _R * _R

    packed = pl.pallas_call(
        _pack_body,
        grid=((n + _B - 1) // _B,),
        in_specs=[
            pl.BlockSpec((_B, _C), lambda i: (i, 0)),
            pl.BlockSpec((_B, _C), lambda i: (i, 0)),
            pl.BlockSpec((_B, 1), lambda i: (i, 0)),
        ],
        out_specs=pl.BlockSpec((_B, 16), lambda i: (i, 0)),
        out_shape=jax.ShapeDtypeStruct((npd, 16), i32),
    )(A, AP, S[:, None])

    mesh = plsc.VectorSubcoreMesh(core_axis_name="c", subcore_axis_name="s")
    sc = pl.kernel(
        functools.partial(_sc_hist, npd),
        out_type=jax.ShapeDtypeStruct((npd, _F), f32),
        mesh=mesh,
        compiler_params=pltpu.CompilerParams(needs_layout_passes=False),
        scratch_types=[
            pltpu.VMEM((_R, 16), i32),
            pltpu.VMEM((_R, _F), f32),
        ],
    )
    feat = sc(packed)

    T = jnp.zeros((_F, 2 * _D), f32)
    T = T.at[0:38, _D:2 * _D].set(atom_table)
    T = T.at[40:56, _D:2 * _D].set(atom_pos_table)
    T = T.at[40 + _PAD, :].set(0.0)
    T = T.at[56:81, 0:_D].set(res_table)
    pos = jnp.arange(256, dtype=f32)[:, None]
    idx = jnp.power(10000.0,
                    -2.0 * jnp.arange(_D // 2, dtype=f32) / _D)[None, :]
    emb = pos * idx
    st = jnp.stack([jnp.sin(emb), jnp.cos(emb)], axis=-1).reshape(256, _D)

    out = pl.pallas_call(
        _tc_body,
        grid=((n + _B - 1) // _B,),
        in_specs=[
            pl.BlockSpec((_B, _F), lambda i: (i, 0)),
            pl.BlockSpec((_B, 1), lambda i: (i, 0)),
            pl.BlockSpec((_F, 2 * _D), lambda i: (0, 0)),
            pl.BlockSpec((256, _D), lambda i: (0, 0)),
        ],
        out_specs=pl.BlockSpec((_B, 2 * _D), lambda i: (i, 0)),
        out_shape=jax.ShapeDtypeStruct((n, 2 * _D), f32),
    )(feat, RP[:, None], T, st)
    return out


# speedup vs baseline: 1.6149x; 1.6149x over previous
import functools

import jax
import jax.numpy as jnp
from jax import lax
from jax.experimental import pallas as pl
from jax.experimental.pallas import tpu as pltpu
from jax.experimental.pallas import tpu_sc as plsc

_C = 14
_D = 64
_PAD = 15
_F = 128
_ZF = 96
_R = 192
_NW = 32
_B = 4096


def _sc_hist(n, x_hbm, feat_hbm, x_v, feat_v):
    f32 = jnp.float32
    i32 = jnp.int32
    nchunks = n // _R
    kmax = (nchunks + _NW - 1) // _NW
    wid = lax.axis_index("s") * 2 + lax.axis_index("c")
    iota = lax.iota(i32, 16)
    ones = jnp.ones((16,), f32)
    zeros = jnp.zeros((16,), f32)


    def chunk_body(k, carry):
        cid = wid + k * _NW

        @pl.when(cid < nchunks)
        def _():
            base = pl.multiple_of(cid * _R, 64)
            pltpu.sync_copy(x_hbm.at[pl.ds(base, _R)], x_v)

            def zbody(r, c2):
                for u in range(_ZF // 16):
                    feat_v[r, pl.ds(u * 16, 16)] = zeros
                return c2

            lax.fori_loop(0, _R, zbody, 0)

            def gbody(g, c2):
                rows0 = g * 16 + iota
                valid = rows0 < _R
                rows = jnp.minimum(rows0, _R - 1)
                col0 = iota * 0
                afs = [plsc.load_gather(x_v, [rows, col0 + c])
                       for c in range(_C)]
                sv = lax.bitwise_and(
                    plsc.load_gather(x_v, [rows, col0 + _C]), 31)
                cnt = zeros
                for c in range(_C):
                    cnt = cnt + jnp.where(
                        lax.bitwise_and(afs[c], 15) != _PAD, 1.0, 0.0)
                recip = 1.0 / (cnt + 1e-10)
                for c in range(_C):
                    ac = lax.bitwise_and(
                        lax.shift_right_logical(afs[c], 4), 63)
                    apc = lax.bitwise_and(afs[c], 15)
                    plsc.addupdate_scatter(feat_v, [rows, ac], recip,
                                           mask=valid & (apc != _PAD))
                    plsc.addupdate_scatter(feat_v, [rows, apc + 40],
                                           recip, mask=valid)
                plsc.addupdate_scatter(feat_v, [rows, sv + 56], ones,
                                       mask=valid)
                return c2

            lax.fori_loop(0, (_R + 15) // 16, gbody, 0)
            pltpu.sync_copy(feat_v, feat_hbm.at[pl.ds(base, _R)])

        return carry

    lax.fori_loop(0, kmax, chunk_body, 0)


def _tc_body(feat_ref, rp_ref, t_ref, st_ref, out_ref):
    f32 = jnp.float32
    full = jnp.dot(feat_ref[...], t_ref[...], preferred_element_type=f32)
    rp = rp_ref[...].astype(f32)
    rpb = jnp.dot(rp, jnp.ones((1, 256), f32), preferred_element_type=f32)
    oh = (rpb == lax.broadcasted_iota(jnp.int32, (1, 256), 1).astype(f32))
    sinrows = jnp.dot(oh.astype(f32), st_ref[...], preferred_element_type=f32)
    out_ref[:, 0:_D] = full[:, 0:_D] + sinrows
    out_ref[:, _D:2 * _D] = full[:, _D:2 * _D]


def kernel(S, RP, A, AP, res_table, atom_table, atom_pos_table):
    f32 = jnp.float32
    i32 = jnp.int32
    n = S.shape[0]
    assert _R % 8 == 0 and (_R // 8) % 8 == 0 and _B % 64 == 0
    npd = (n + _R - 1) // _R * _R

    af = A * 16 + AP
    packed = jnp.concatenate(
        [af, S[:, None], jnp.zeros((n, 1), i32)], axis=1)
    packed = jnp.pad(packed, ((0, npd - n), (0, 0)))

    mesh = plsc.VectorSubcoreMesh(core_axis_name="c", subcore_axis_name="s")
    sc = pl.kernel(
        functools.partial(_sc_hist, npd),
        out_type=jax.ShapeDtypeStruct((npd, _F), f32),
        mesh=mesh,
        compiler_params=pltpu.CompilerParams(needs_layout_passes=False),
        scratch_types=[
            pltpu.VMEM((_R, 16), i32),
            pltpu.VMEM((_R, _F), f32),
        ],
    )
    feat = sc(packed)

    T = jnp.zeros((_F, 2 * _D), f32)
    T = T.at[0:38, _D:2 * _D].set(atom_table)
    T = T.at[40:56, _D:2 * _D].set(atom_pos_table)
    T = T.at[40 + _PAD, :].set(0.0)
    T = T.at[56:81, 0:_D].set(res_table)
    pos = jnp.arange(256, dtype=f32)[:, None]
    idx = jnp.power(10000.0,
                    -2.0 * jnp.arange(_D // 2, dtype=f32) / _D)[None, :]
    emb = pos * idx
    st = jnp.stack([jnp.sin(emb), jnp.cos(emb)], axis=-1).reshape(256, _D)

    out = pl.pallas_call(
        _tc_body,
        grid=((n + _B - 1) // _B,),
        in_specs=[
            pl.BlockSpec((_B, _F), lambda i: (i, 0)),
            pl.BlockSpec((_B, 1), lambda i: (i, 0)),
            pl.BlockSpec((_F, 2 * _D), lambda i: (0, 0)),
            pl.BlockSpec((256, _D), lambda i: (0, 0)),
        ],
        out_specs=pl.BlockSpec((_B, 2 * _D), lambda i: (i, 0)),
        out_shape=jax.ShapeDtypeStruct((n, 2 * _D), f32),
    )(feat, RP[:, None], T, st)
    return out


# speedup vs baseline: 1.6710x; 1.0347x over previous
import functools

import jax
import jax.numpy as jnp
from jax import lax
from jax.experimental import pallas as pl
from jax.experimental.pallas import tpu as pltpu
from jax.experimental.pallas import tpu_sc as plsc

_C = 14
_D = 64
_PAD = 15
_F = 128
_ZF = 96
_R = 384
_NW = 32
_B = 4096


def _sc_hist(n, x_hbm, feat_hbm, x_v, feat_v):
    f32 = jnp.float32
    i32 = jnp.int32
    nchunks = n // _R
    kmax = (nchunks + _NW - 1) // _NW
    wid = lax.axis_index("s") * 2 + lax.axis_index("c")
    iota = lax.iota(i32, 16)
    ones = jnp.ones((16,), f32)
    zeros = jnp.zeros((16,), f32)


    def chunk_body(k, carry):
        cid = wid + k * _NW

        @pl.when(cid < nchunks)
        def _():
            base = pl.multiple_of(cid * _R, 64)
            pltpu.sync_copy(x_hbm.at[pl.ds(base, _R)], x_v)

            def zbody(r, c2):
                for u in range(_ZF // 16):
                    feat_v[r, pl.ds(u * 16, 16)] = zeros
                return c2

            lax.fori_loop(0, _R, zbody, 0)

            def gbody(g, c2):
                rows0 = g * 16 + iota
                valid = rows0 < _R
                rows = jnp.minimum(rows0, _R - 1)
                col0 = iota * 0
                afs = [plsc.load_gather(x_v, [rows, col0 + c])
                       for c in range(_C)]
                sv = lax.bitwise_and(
                    plsc.load_gather(x_v, [rows, col0 + _C]), 31)
                cnt = zeros
                for c in range(_C):
                    cnt = cnt + jnp.where(
                        lax.bitwise_and(afs[c], 15) != _PAD, 1.0, 0.0)
                recip = 1.0 / (cnt + 1e-10)
                for c in range(_C):
                    ac = lax.bitwise_and(
                        lax.shift_right_logical(afs[c], 4), 63)
                    apc = lax.bitwise_and(afs[c], 15)
                    plsc.addupdate_scatter(feat_v, [rows, ac], recip,
                                           mask=valid & (apc != _PAD))
                    plsc.addupdate_scatter(feat_v, [rows, apc + 40],
                                           recip, mask=valid)
                plsc.addupdate_scatter(feat_v, [rows, sv + 56], ones,
                                       mask=valid)
                return c2

            lax.fori_loop(0, (_R + 15) // 16, gbody, 0)
            pltpu.sync_copy(feat_v, feat_hbm.at[pl.ds(base, _R)])

        return carry

    lax.fori_loop(0, kmax, chunk_body, 0)


def _tc_body(feat_ref, rp_ref, t_ref, st_ref, out_ref):
    f32 = jnp.float32
    full = jnp.dot(feat_ref[...], t_ref[...], preferred_element_type=f32)
    rp = rp_ref[...].astype(f32)
    rpb = jnp.dot(rp, jnp.ones((1, 256), f32), preferred_element_type=f32)
    oh = (rpb == lax.broadcasted_iota(jnp.int32, (1, 256), 1).astype(f32))
    sinrows = jnp.dot(oh.astype(f32), st_ref[...], preferred_element_type=f32)
    out_ref[:, 0:_D] = full[:, 0:_D] + sinrows
    out_ref[:, _D:2 * _D] = full[:, _D:2 * _D]


def kernel(S, RP, A, AP, res_table, atom_table, atom_pos_table):
    f32 = jnp.float32
    i32 = jnp.int32
    n = S.shape[0]
    assert _R % 8 == 0 and (_R // 8) % 8 == 0 and _B % 64 == 0
    npd = (n + _R - 1) // _R * _R

    af = A * 16 + AP
    packed = jnp.concatenate(
        [af, S[:, None], jnp.zeros((n, 1), i32)], axis=1)
    packed = jnp.pad(packed, ((0, npd - n), (0, 0)))

    mesh = plsc.VectorSubcoreMesh(core_axis_name="c", subcore_axis_name="s")
    sc = pl.kernel(
        functools.partial(_sc_hist, npd),
        out_type=jax.ShapeDtypeStruct((npd, _F), f32),
        mesh=mesh,
        compiler_params=pltpu.CompilerParams(needs_layout_passes=False),
        scratch_types=[
            pltpu.VMEM((_R, 16), i32),
            pltpu.VMEM((_R, _F), f32),
        ],
    )
    feat = sc(packed)

    T = jnp.zeros((_F, 2 * _D), f32)
    T = T.at[0:38, _D:2 * _D].set(atom_table)
    T = T.at[40:56, _D:2 * _D].set(atom_pos_table)
    T = T.at[40 + _PAD, :].set(0.0)
    T = T.at[56:81, 0:_D].set(res_table)
    pos = jnp.arange(256, dtype=f32)[:, None]
    idx = jnp.power(10000.0,
                    -2.0 * jnp.arange(_D // 2, dtype=f32) / _D)[None, :]
    emb = pos * idx
    st = jnp.stack([jnp.sin(emb), jnp.cos(emb)], axis=-1).reshape(256, _D)

    out = pl.pallas_call(
        _tc_body,
        grid=((n + _B - 1) // _B,),
        in_specs=[
            pl.BlockSpec((_B, _F), lambda i: (i, 0)),
            pl.BlockSpec((_B, 1), lambda i: (i, 0)),
            pl.BlockSpec((_F, 2 * _D), lambda i: (0, 0)),
            pl.BlockSpec((256, _D), lambda i: (0, 0)),
        ],
        out_specs=pl.BlockSpec((_B, 2 * _D), lambda i: (i, 0)),
        out_shape=jax.ShapeDtypeStruct((n, 2 * _D), f32),
    )(feat, RP[:, None], T, st)
    return out
